# Initial kernel scaffold; baseline (speedup 1.0000x reference)
import functools

import jax
import jax.numpy as jnp
from jax import lax
from jax.experimental import pallas as pl
from jax.experimental.pallas import tpu as pltpu
from jax.experimental.pallas import tpu_sc as plsc

N = 10000
E = 320000
D = 128
EXT = 144
NC = 2
NS = 16
NW = NC * NS
EPW = E // NW
C = 80
NCHUNK = EPW // C
GRP = C // 16
RPT = N // NS
ZR = 125


def _inv_norm_body(feat_ref, inv_ref):
    x = feat_ref[...]
    ss = jnp.sum(x * x, axis=1)
    inv_ref[...] = 1.0 / jnp.maximum(jnp.sqrt(ss), 1e-12)


def _inv_norms(feat):
    return pl.pallas_call(
        _inv_norm_body,
        out_shape=jax.ShapeDtypeStruct((N,), jnp.float32),
    )(feat)


def _finalize_body(parts_ref, out_ref):
    ext = parts_ref[0] + parts_ref[1]
    den = jnp.maximum(ext[:, D:D + 1], 1e-30)
    out_ref[...] = ext[:, :D] / den


def _finalize(parts):
    return pl.pallas_call(
        _finalize_body,
        out_shape=jax.ShapeDtypeStruct((N, D), jnp.float32),
    )(parts)


def _sc_body(feat_hbm, inv_hbm, invb_hbm, src_hbm, dst_hbm, out_hbm,
             acc_sh, src_idx, dst_idx, feat_s, feat_d, msg,
             inv_t, invb_t, srow, wbuf, zbuf, sem):
    cid = lax.axis_index("c")
    sid = lax.axis_index("s")
    wid = cid * NS + sid

    def zrow(r, _):
        for j in range(EXT // 16):
            zbuf[r, pl.ds(j * 16, 16)] = jnp.zeros((16,), jnp.float32)
        return 0
    lax.fori_loop(0, ZR, zrow, 0)
    for k in range(RPT // ZR):
        pltpu.sync_copy(zbuf, acc_sh.at[pl.ds(sid * RPT + k * ZR, ZR)])

    def zmsg(r, _):
        msg[r, pl.ds(D, 16)] = jnp.zeros((16,), jnp.float32)
        return 0
    lax.fori_loop(0, C, zmsg, 0)

    pltpu.sync_copy(inv_hbm, inv_t)
    pltpu.sync_copy(invb_hbm, invb_t)
    plsc.subcore_barrier()

    rows16 = lax.broadcasted_iota(jnp.int32, (16,), 0)
    base0 = wid * EPW

    def chunk(ci, _):
        base = base0 + ci * C
        pltpu.sync_copy(src_hbm.at[pl.ds(base, C)], src_idx)
        pltpu.sync_copy(dst_hbm.at[pl.ds(base, C)], dst_idx)
        pltpu.async_copy(feat_hbm.at[src_idx], feat_s, sem).wait()
        pltpu.async_copy(feat_hbm.at[dst_idx], feat_d, sem).wait()
        for g in range(GRP):
            e0 = g * 16
            for e in range(16):
                acc = feat_s[e0 + e, pl.ds(0, 16)] * feat_d[e0 + e, pl.ds(0, 16)]
                for j in range(1, D // 16):
                    acc = acc + (feat_s[e0 + e, pl.ds(j * 16, 16)]
                                 * feat_d[e0 + e, pl.ds(j * 16, 16)])
                srow[e, :] = acc
            tot = plsc.load_gather(srow, [rows16, jnp.zeros((16,), jnp.int32)])
            for j in range(1, 16):
                tot = tot + plsc.load_gather(
                    srow, [rows16, jnp.full((16,), j, jnp.int32)])
            sidx = src_idx[pl.ds(e0, 16)]
            didx = dst_idx[pl.ds(e0, 16)]
            ivb = plsc.load_gather(invb_t, [sidx])
            iv = plsc.load_gather(inv_t, [didx])
            w = jnp.exp(tot * ivb * iv)
            wbuf[...] = w
            plsc.store_scatter(msg, [rows16 + e0, jnp.full((16,), D, jnp.int32)], w)
            for e in range(16):
                ws = wbuf[e]
                for j in range(D // 16):
                    msg[e0 + e, pl.ds(j * 16, 16)] = (
                        feat_s[e0 + e, pl.ds(j * 16, 16)] * ws)
        pltpu.sync_copy(msg, acc_sh.at[dst_idx], add=True)
        return 0

    lax.fori_loop(0, NCHUNK, chunk, 0)
    plsc.subcore_barrier()

    for k in range(RPT // ZR):
        r0 = sid * RPT + k * ZR
        pltpu.sync_copy(acc_sh.at[pl.ds(r0, ZR)], zbuf)
        pltpu.sync_copy(zbuf, out_hbm.at[cid, pl.ds(r0, ZR)])


def _sc_edge_pass(feat, inv, invb, src, dst):
    mesh = plsc.VectorSubcoreMesh(core_axis_name="c", subcore_axis_name="s")
    return pl.kernel(
        _sc_body,
        out_type=jax.ShapeDtypeStruct((NC, N, EXT), jnp.float32),
        mesh=mesh,
        scratch_types=[
            pltpu.VMEM_SHARED((N, EXT), jnp.float32),
            pltpu.VMEM((C,), jnp.int32),
            pltpu.VMEM((C,), jnp.int32),
            pltpu.VMEM((C, D), jnp.float32),
            pltpu.VMEM((C, D), jnp.float32),
            pltpu.VMEM((C, EXT), jnp.float32),
            pltpu.VMEM((N,), jnp.float32),
            pltpu.VMEM((N,), jnp.float32),
            pltpu.VMEM((16, 16), jnp.float32),
            pltpu.VMEM((16,), jnp.float32),
            pltpu.VMEM((ZR, EXT), jnp.float32),
            pltpu.SemaphoreType.DMA,
        ],
    )(feat, inv, invb, src, dst)


def kernel(feat, edge_index, beta):
    src = edge_index[0].astype(jnp.int32)
    dst = edge_index[1].astype(jnp.int32)
    inv = _inv_norms(feat)
    invb = inv * beta
    parts = _sc_edge_pass(feat, inv, invb, src, dst)
    return _finalize(parts)


# speedup vs baseline: 6.1707x; 6.1707x over previous
import jax
import jax.numpy as jnp
from jax import lax
from jax.experimental import pallas as pl
from jax.experimental.pallas import tpu as pltpu
from jax.experimental.pallas import tpu_sc as plsc

N = 10000
E = 320000
D = 128
EXT = 144
NC = 2
NS = 16
NW = NC * NS
EPW = E // NW
C = 80
NCHUNK = EPW // C
GRP = C // 16
RPT = N // NS
ZR = 25


def _prep_body(beta_ref, feat_ref, ext_ref):
    x = feat_ref[...]
    ss = jnp.sum(x * x, axis=1, keepdims=True)
    inv = 1.0 / jnp.maximum(jnp.sqrt(ss), 1e-12)
    pad = jnp.zeros((N, EXT - D - 2), jnp.float32)
    ext_ref[...] = jnp.concatenate([x, inv, inv * beta_ref[0, 0], pad], axis=1)


def _prep(feat, beta):
    return pl.pallas_call(
        _prep_body,
        in_specs=[
            pl.BlockSpec(memory_space=pltpu.SMEM),
            pl.BlockSpec(memory_space=pltpu.VMEM),
        ],
        out_shape=jax.ShapeDtypeStruct((N, EXT), jnp.float32),
    )(jnp.reshape(beta, (1, 1)), feat)


def _finalize_body(parts_ref, out_ref):
    ext = parts_ref[0] + parts_ref[1]
    den = jnp.maximum(ext[:, D:D + 1], 1e-30)
    out_ref[...] = ext[:, :D] / den


def _finalize(parts):
    return pl.pallas_call(
        _finalize_body,
        out_shape=jax.ShapeDtypeStruct((N, D), jnp.float32),
    )(parts)


def _sc_body(ext_hbm, src_hbm, dst_hbm, out_hbm,
             acc_sh, src_idx, dst_idx, feat_s, feat_d, srow, sem):
    cid = lax.axis_index("c")
    sid = lax.axis_index("s")
    wid = cid * NS + sid

    def zrow(r, _):
        for j in range(EXT // 16):
            feat_s[r, pl.ds(j * 16, 16)] = jnp.zeros((16,), jnp.float32)
        return 0
    lax.fori_loop(0, ZR, zrow, 0)

    def zcopy(k, _):
        pltpu.sync_copy(feat_s.at[pl.ds(0, ZR)],
                        acc_sh.at[pl.ds(sid * RPT + k * ZR, ZR)])
        return 0
    lax.fori_loop(0, RPT // ZR, zcopy, 0)
    plsc.subcore_barrier()

    rows16 = lax.broadcasted_iota(jnp.int32, (16,), 0)
    base0 = wid * EPW

    def chunk(ci, _):
        base = base0 + ci * C
        pltpu.sync_copy(src_hbm.at[pl.ds(base, C)], src_idx)
        pltpu.sync_copy(dst_hbm.at[pl.ds(base, C)], dst_idx)
        pltpu.async_copy(ext_hbm.at[src_idx], feat_s, sem).wait()
        pltpu.async_copy(ext_hbm.at[dst_idx], feat_d, sem).wait()
        for g in range(GRP):
            e0 = g * 16
            for e in range(16):
                acc = feat_s[e0 + e, pl.ds(0, 16)] * feat_d[e0 + e, pl.ds(0, 16)]
                for j in range(1, D // 16):
                    acc = acc + (feat_s[e0 + e, pl.ds(j * 16, 16)]
                                 * feat_d[e0 + e, pl.ds(j * 16, 16)])
                srow[e, :] = acc
            tot = plsc.load_gather(srow, [rows16, jnp.zeros((16,), jnp.int32)])
            for j in range(1, 16):
                tot = tot + plsc.load_gather(
                    srow, [rows16, jnp.full((16,), j, jnp.int32)])
            ivb = plsc.load_gather(
                feat_s, [rows16 + e0, jnp.full((16,), D + 1, jnp.int32)])
            iv = plsc.load_gather(
                feat_d, [rows16 + e0, jnp.full((16,), D, jnp.int32)])
            w = jnp.exp(tot * ivb * iv)
            for e in range(16):
                ws = w[e]
                for j in range(D // 16):
                    feat_s[e0 + e, pl.ds(j * 16, 16)] = (
                        feat_s[e0 + e, pl.ds(j * 16, 16)] * ws)
                feat_s[e0 + e, pl.ds(D, 16)] = jnp.zeros((16,), jnp.float32)
            plsc.store_scatter(
                feat_s, [rows16 + e0, jnp.full((16,), D, jnp.int32)], w)
        pltpu.sync_copy(feat_s, acc_sh.at[dst_idx], add=True)
        return 0

    lax.fori_loop(0, NCHUNK, chunk, 0)
    plsc.subcore_barrier()

    def ocopy(k, _):
        r0 = sid * RPT + k * ZR
        pltpu.sync_copy(acc_sh.at[pl.ds(r0, ZR)], feat_s.at[pl.ds(0, ZR)])
        pltpu.sync_copy(feat_s.at[pl.ds(0, ZR)], out_hbm.at[cid, pl.ds(r0, ZR)])
        return 0
    lax.fori_loop(0, RPT // ZR, ocopy, 0)


def _sc_edge_pass(ext, src, dst):
    mesh = plsc.VectorSubcoreMesh(core_axis_name="c", subcore_axis_name="s")
    return pl.kernel(
        _sc_body,
        out_type=jax.ShapeDtypeStruct((NC, N, EXT), jnp.float32),
        mesh=mesh,
        compiler_params=pltpu.CompilerParams(
            use_tc_tiling_on_sc=False, needs_layout_passes=False),
        scratch_types=[
            pltpu.VMEM_SHARED((N, EXT), jnp.float32),
            pltpu.VMEM((C,), jnp.int32),
            pltpu.VMEM((C,), jnp.int32),
            pltpu.VMEM((C, EXT), jnp.float32),
            pltpu.VMEM((C, EXT), jnp.float32),
            pltpu.VMEM((16, 16), jnp.float32),
            pltpu.SemaphoreType.DMA,
        ],
    )(ext, src, dst)


def kernel(feat, edge_index, beta):
    src = edge_index[0].astype(jnp.int32)
    dst = edge_index[1].astype(jnp.int32)
    ext = _prep(feat, beta.astype(jnp.float32))
    parts = _sc_edge_pass(ext, src, dst)
    return _finalize(parts)


# speedup vs baseline: 8.5529x; 1.3861x over previous
import jax
import jax.numpy as jnp
from jax import lax
from jax.experimental import pallas as pl
from jax.experimental.pallas import tpu as pltpu
from jax.experimental.pallas import tpu_sc as plsc

N = 10000
E = 320000
D = 128
EXT = 144
NC = 2
NS = 16
NW = NC * NS
C = 32
NCH = 315
EPW = NCH * C
EP = NW * EPW
GRP = C // 16
RPT = N // NS
ZR = 25


def _prep_body(beta_ref, feat_ref, ext_ref):
    x = feat_ref[...]
    ss = jnp.sum(x * x, axis=1, keepdims=True)
    inv = 1.0 / jnp.maximum(jnp.sqrt(ss), 1e-12)
    pad = jnp.zeros((N, EXT - D - 2), jnp.float32)
    ext_ref[...] = jnp.concatenate([x, inv, inv * beta_ref[0, 0], pad], axis=1)


def _prep(feat, beta):
    return pl.pallas_call(
        _prep_body,
        in_specs=[
            pl.BlockSpec(memory_space=pltpu.SMEM),
            pl.BlockSpec(memory_space=pltpu.VMEM),
        ],
        out_shape=jax.ShapeDtypeStruct((N, EXT), jnp.float32),
    )(jnp.reshape(beta, (1, 1)), feat)


def _finalize_body(parts_ref, out_ref):
    ext = parts_ref[0] + parts_ref[1]
    den = jnp.maximum(ext[:, D:D + 1], 1e-30)
    out_ref[...] = ext[:, :D] / den


def _finalize(parts):
    return pl.pallas_call(
        _finalize_body,
        out_shape=jax.ShapeDtypeStruct((N, D), jnp.float32),
    )(parts)


def _sc_body(ext_hbm, sd_hbm, out_hbm, acc_sh,
             idx0, idx1, idx2, sx0, sx1, sx2, fs0, fs1, fs2, fd0, fd1, fd2,
             srow, is0, is1, is2, gs0, gs1, gs2, ss0, ss1, ss2):
    cid = lax.axis_index("c")
    sid = lax.axis_index("s")
    wid = cid * NS + sid
    idxs = [idx0, idx1, idx2]
    sidx = [sx0, sx1, sx2]
    fss = [fs0, fs1, fs2]
    fds = [fd0, fd1, fd2]
    isems = [is0, is1, is2]
    gsems = [gs0, gs1, gs2]
    ssems = [ss0, ss1, ss2]

    def zrow(r, _):
        for j in range(EXT // 16):
            fs0[r, pl.ds(j * 16, 16)] = jnp.zeros((16,), jnp.float32)
        return 0
    lax.fori_loop(0, ZR, zrow, 0)

    def zcopy(k, _):
        pltpu.sync_copy(fs0.at[pl.ds(0, ZR)],
                        acc_sh.at[pl.ds(sid * RPT + k * ZR, ZR)])
        return 0
    lax.fori_loop(0, RPT // ZR, zcopy, 0)
    plsc.subcore_barrier()

    rows16 = lax.broadcasted_iota(jnp.int32, (16,), 0)
    chunk0 = wid * NCH
    ebase0 = wid * EPW

    def stage_idx(p, s):
        pltpu.async_copy(sd_hbm.at[chunk0 + p], idxs[s], isems[s])

    def issue_gather(p, s):
        pltpu.make_async_copy(sd_hbm.at[chunk0 + p], idxs[s], isems[s]).wait()
        pltpu.async_copy(ext_hbm.at[idxs[s].at[0]], fss[s], gsems[s])
        pltpu.async_copy(ext_hbm.at[idxs[s].at[1]], fds[s], gsems[s])

    def wait_gather(s):
        pltpu.make_async_copy(ext_hbm.at[idxs[s].at[0]], fss[s], gsems[s]).wait()
        pltpu.make_async_copy(ext_hbm.at[idxs[s].at[1]], fds[s], gsems[s]).wait()

    def issue_scatter(s):
        pltpu.async_copy(fss[s], acc_sh.at[sidx[s]], ssems[s], add=True)

    def wait_scatter(s):
        pltpu.make_async_copy(fss[s], acc_sh.at[sidx[s]], ssems[s]).wait()

    def compute(p, s):
        fs, fd = fss[s], fds[s]
        for g in range(GRP):
            e0 = g * 16
            for e in range(16):
                acc = fs[e0 + e, pl.ds(0, 16)] * fd[e0 + e, pl.ds(0, 16)]
                for j in range(1, D // 16):
                    acc = acc + (fs[e0 + e, pl.ds(j * 16, 16)]
                                 * fd[e0 + e, pl.ds(j * 16, 16)])
                srow[e, :] = acc
            tot = plsc.load_gather(srow, [rows16, jnp.zeros((16,), jnp.int32)])
            for j in range(1, 16):
                tot = tot + plsc.load_gather(
                    srow, [rows16, jnp.full((16,), j, jnp.int32)])
            ivb = plsc.load_gather(
                fs, [rows16 + e0, jnp.full((16,), D + 1, jnp.int32)])
            iv = plsc.load_gather(
                fd, [rows16 + e0, jnp.full((16,), D, jnp.int32)])
            w = jnp.exp(tot * ivb * iv)
            gidx = ebase0 + p * C + e0 + rows16
            w = jnp.where(gidx < E, w, 0.0)
            for e in range(16):
                ws = w[e]
                for j in range(D // 16):
                    fs[e0 + e, pl.ds(j * 16, 16)] = (
                        fs[e0 + e, pl.ds(j * 16, 16)] * ws)
                fs[e0 + e, pl.ds(D, 16)] = jnp.zeros((16,), jnp.float32)
            plsc.store_scatter(
                fs, [rows16 + e0, jnp.full((16,), D, jnp.int32)], w)

    for p0 in range(3):
        stage_idx(p0, p0)
    issue_gather(0, 0)
    issue_gather(1, 1)

    KMAX = NCH // 3

    def body(k, _):
        for r in range(3):
            s = r
            p = 3 * k + r
            wait_gather(s)
            for j in range(C // 16):
                sidx[s][pl.ds(j * 16, 16)] = idxs[s][1, pl.ds(j * 16, 16)]
            @pl.when(k < KMAX - 1)
            def _():
                stage_idx(p + 3, s)
            compute(p, s)
            issue_scatter(s)
            nxt = (r + 2) % 3
            if r == 0:
                @pl.when(k > 0)
                def _():
                    wait_scatter(nxt)
                issue_gather(p + 2, nxt)
            else:
                wait_scatter(nxt)
                @pl.when(k < KMAX - 1)
                def _():
                    issue_gather(p + 2, nxt)
        return 0

    lax.fori_loop(0, KMAX, body, 0)
    wait_scatter(2)
    plsc.subcore_barrier()

    def ocopy(k, _):
        r0 = sid * RPT + k * ZR
        pltpu.sync_copy(acc_sh.at[pl.ds(r0, ZR)], fs0.at[pl.ds(0, ZR)])
        pltpu.sync_copy(fs0.at[pl.ds(0, ZR)], out_hbm.at[cid, pl.ds(r0, ZR)])
        return 0
    lax.fori_loop(0, RPT // ZR, ocopy, 0)


def _sc_edge_pass(ext, sd):
    mesh = plsc.VectorSubcoreMesh(core_axis_name="c", subcore_axis_name="s")
    return pl.kernel(
        _sc_body,
        out_type=jax.ShapeDtypeStruct((NC, N, EXT), jnp.float32),
        mesh=mesh,
        compiler_params=pltpu.CompilerParams(
            use_tc_tiling_on_sc=False, needs_layout_passes=False),
        scratch_types=(
            [pltpu.VMEM_SHARED((N, EXT), jnp.float32)]
            + [pltpu.VMEM((2, C), jnp.int32)] * 3
            + [pltpu.VMEM((C,), jnp.int32)] * 3
            + [pltpu.VMEM((C, EXT), jnp.float32)] * 6
            + [pltpu.VMEM((16, 16), jnp.float32)]
            + [pltpu.SemaphoreType.DMA] * 9
        ),
    )(ext, sd)


def kernel(feat, edge_index, beta):
    src = edge_index[0].astype(jnp.int32)
    dst = edge_index[1].astype(jnp.int32)
    padn = EP - E
    srcp = jnp.concatenate([src, jnp.zeros((padn,), jnp.int32)])
    dstp = jnp.concatenate([dst, jnp.zeros((padn,), jnp.int32)])
    sd = jnp.stack([srcp.reshape(-1, C), dstp.reshape(-1, C)], axis=1)
    ext = _prep(feat, beta.astype(jnp.float32))
    parts = _sc_edge_pass(ext, sd)
    return _finalize(parts)


# speedup vs baseline: 11.8083x; 1.3806x over previous
import numpy as np

import jax
import jax.numpy as jnp
from jax import lax
from jax.experimental import pallas as pl
from jax.experimental.pallas import tpu as pltpu
from jax.experimental.pallas import tpu_sc as plsc

N = 10000
E = 320000
D = 128
TW = 160
EXT = 144
NC = 2
NS = 16
NW = NC * NS
C = 48
NCH = 210
EPW = NCH * C
EP = NW * EPW
GRP = C // 16
RPT = N // NS
ZR = 25

_PIN = np.empty((D,), np.int64)
for _j in range(4):
    for _i in range(16):
        _PIN[32 * _j + 2 * _i] = 32 * _j + _i
        _PIN[32 * _j + 2 * _i + 1] = 32 * _j + 16 + _i
_PIN = tuple(int(x) for x in _PIN)


def _prep_body(beta_ref, feat_ref, tbl_ref):
    x = feat_ref[...]
    ss = jnp.sum(x * x, axis=1, keepdims=True)
    nmax = jnp.maximum(jnp.sqrt(ss), 1e-12)
    nh = x / nmax
    b = jnp.full((N, 1), beta_ref[0, 0], jnp.float32)
    pad = jnp.zeros((N, TW - D - 4), jnp.float32)
    row = jnp.concatenate([nh, nmax, nmax, b, b, pad], axis=1)
    tbl_ref[...] = row.astype(jnp.bfloat16)


def _prep(featp, beta):
    return pl.pallas_call(
        _prep_body,
        in_specs=[
            pl.BlockSpec(memory_space=pltpu.SMEM),
            pl.BlockSpec(memory_space=pltpu.VMEM),
        ],
        out_shape=jax.ShapeDtypeStruct((N, TW), jnp.bfloat16),
    )(jnp.reshape(beta, (1, 1)), featp)


def _finalize_body(parts_ref, out_ref):
    ext = parts_ref[0] + parts_ref[1]
    den = jnp.maximum(ext[:, D:D + 1], 1e-30)
    out_ref[...] = ext[:, :D] / den


def _finalize(parts):
    return pl.pallas_call(
        _finalize_body,
        out_shape=jax.ShapeDtypeStruct((N, D), jnp.float32),
    )(parts)


def _sc_body(tbl_hbm, sd_hbm, out_hbm, acc_sh,
             idx0, idx1, sx0, sx1, fs0, fs1, fd0, fd1, mg0, mg1,
             srow, is0, is1, gs0, gs1, ss0, ss1):
    cid = lax.axis_index("c")
    sid = lax.axis_index("s")
    wid = cid * NS + sid
    idxs = [idx0, idx1]
    sidx = [sx0, sx1]
    fss = [fs0, fs1]
    fds = [fd0, fd1]
    msgs = [mg0, mg1]
    isems = [is0, is1]
    gsems = [gs0, gs1]
    ssems = [ss0, ss1]

    def zrow(r, _):
        for j in range(EXT // 16):
            mg0[r, pl.ds(j * 16, 16)] = jnp.zeros((16,), jnp.float32)
        return 0
    lax.fori_loop(0, ZR, zrow, 0)

    def zcopy(k, _):
        pltpu.sync_copy(mg0.at[pl.ds(0, ZR)],
                        acc_sh.at[pl.ds(sid * RPT + k * ZR, ZR)])
        return 0
    lax.fori_loop(0, RPT // ZR, zcopy, 0)

    def zmsg(r, _):
        mg0[r, pl.ds(D, 16)] = jnp.zeros((16,), jnp.float32)
        mg1[r, pl.ds(D, 16)] = jnp.zeros((16,), jnp.float32)
        return 0
    lax.fori_loop(0, C, zmsg, 0)
    plsc.subcore_barrier()

    rows16 = lax.broadcasted_iota(jnp.int32, (16,), 0)
    chunk0 = wid * NCH
    ebase0 = wid * EPW

    def stage_idx(p, s):
        pltpu.async_copy(sd_hbm.at[chunk0 + p], idxs[s], isems[s])

    def issue_gather(p, s):
        pltpu.make_async_copy(sd_hbm.at[chunk0 + p], idxs[s], isems[s]).wait()
        pltpu.async_copy(tbl_hbm.at[idxs[s].at[0]], fss[s], gsems[s])
        pltpu.async_copy(tbl_hbm.at[idxs[s].at[1]], fds[s], gsems[s])

    def wait_gather(s):
        pltpu.make_async_copy(tbl_hbm.at[idxs[s].at[0]], fss[s], gsems[s]).wait()
        pltpu.make_async_copy(tbl_hbm.at[idxs[s].at[1]], fds[s], gsems[s]).wait()

    def issue_scatter(s):
        pltpu.async_copy(msgs[s], acc_sh.at[sidx[s]], ssems[s], add=True)

    def wait_scatter(s):
        pltpu.make_async_copy(msgs[s], acc_sh.at[sidx[s]], ssems[s]).wait()

    def unpack2(v):
        return plsc.unpack(v, format=plsc.PackFormat.INTERLEAVED,
                           preferred_element_type=jnp.float32)

    def compute(p, s):
        fs, fd, msg = fss[s], fds[s], msgs[s]
        bscal = None
        for g in range(GRP):
            e0 = g * 16
            nm = [None] * 16
            for e in range(16):
                row = e0 + e
                dp = None
                for j in range(D // 32):
                    qa, qb = unpack2(fs[row, pl.ds(j * 32, 32)])
                    ta, tb = unpack2(fd[row, pl.ds(j * 32, 32)])
                    term = qa * ta + qb * tb
                    dp = term if dp is None else dp + term
                srow[e, :] = dp
                sa, _sb = unpack2(fs[row, pl.ds(D, 32)])
                nm[e] = sa[0]
                if bscal is None:
                    bscal = sa[1]
            tot = plsc.load_gather(srow, [rows16, jnp.zeros((16,), jnp.int32)])
            for j in range(1, 16):
                tot = tot + plsc.load_gather(
                    srow, [rows16, jnp.full((16,), j, jnp.int32)])
            w = jnp.exp(tot * bscal)
            gidx = ebase0 + p * C + e0 + rows16
            w = jnp.where(gidx < E, w, 0.0)
            for e in range(16):
                row = e0 + e
                ws2 = w[e] * nm[e]
                for j in range(D // 32):
                    qa, qb = unpack2(fs[row, pl.ds(j * 32, 32)])
                    msg[row, pl.ds(j * 32, 16)] = qa * ws2
                    msg[row, pl.ds(j * 32 + 16, 16)] = qb * ws2
            plsc.store_scatter(
                msg, [rows16 + e0, jnp.full((16,), D, jnp.int32)], w)

    stage_idx(0, 0)
    stage_idx(1, 1)
    issue_gather(0, 0)
    issue_gather(1, 1)

    KMAX = NCH // 2

    def body(k, _):
        for r in range(2):
            s = r
            p = 2 * k + r
            wait_gather(s)
            for j in range(C // 16):
                sidx[s][pl.ds(j * 16, 16)] = idxs[s][1, pl.ds(j * 16, 16)]
            @pl.when(k < KMAX - 1)
            def _():
                stage_idx(p + 2, s)
            compute(p, s)
            if r == 0:
                @pl.when(k > 0)
                def _():
                    wait_scatter(1)
            else:
                wait_scatter(0)
            issue_scatter(s)
            @pl.when(k < KMAX - 1)
            def _():
                issue_gather(p + 2, s)
        return 0

    lax.fori_loop(0, KMAX, body, 0)
    wait_scatter(1)
    plsc.subcore_barrier()

    def ocopy(k, _):
        r0 = sid * RPT + k * ZR
        pltpu.sync_copy(acc_sh.at[pl.ds(r0, ZR)], mg0.at[pl.ds(0, ZR)])
        pltpu.sync_copy(mg0.at[pl.ds(0, ZR)], out_hbm.at[cid, pl.ds(r0, ZR)])
        return 0
    lax.fori_loop(0, RPT // ZR, ocopy, 0)


def _sc_edge_pass(tbl, sd):
    mesh = plsc.VectorSubcoreMesh(core_axis_name="c", subcore_axis_name="s")
    return pl.kernel(
        _sc_body,
        out_type=jax.ShapeDtypeStruct((NC, N, EXT), jnp.float32),
        mesh=mesh,
        compiler_params=pltpu.CompilerParams(
            use_tc_tiling_on_sc=False, needs_layout_passes=False),
        scratch_types=(
            [pltpu.VMEM_SHARED((N, EXT), jnp.float32)]
            + [pltpu.VMEM((2, C), jnp.int32)] * 2
            + [pltpu.VMEM((C,), jnp.int32)] * 2
            + [pltpu.VMEM((C, TW), jnp.bfloat16)] * 4
            + [pltpu.VMEM((C, EXT), jnp.float32)] * 2
            + [pltpu.VMEM((16, 16), jnp.float32)]
            + [pltpu.SemaphoreType.DMA] * 6
        ),
    )(tbl, sd)


def kernel(feat, edge_index, beta):
    src = edge_index[0].astype(jnp.int32)
    dst = edge_index[1].astype(jnp.int32)
    padn = EP - E
    srcp = jnp.concatenate([src, jnp.zeros((padn,), jnp.int32)])
    dstp = jnp.concatenate([dst, jnp.zeros((padn,), jnp.int32)])
    sd = jnp.stack([srcp.reshape(-1, C), dstp.reshape(-1, C)], axis=1)
    featp = feat[:, list(_PIN)]
    tbl = _prep(featp, beta.astype(jnp.float32))
    parts = _sc_edge_pass(tbl, sd)
    return _finalize(parts)
